# initial kernel scaffold (unmeasured)
import jax
import jax.numpy as jnp
from jax import lax
from jax.experimental import pallas as pl
from jax.experimental.pallas import tpu as pltpu

N_DEV = 4
N_LAYERS = 3


def kernel(x, Win0, Wout0, Win1, Wout1, Win2, Wout2):
    b, d = x.shape
    rows_per = b // N_DEV

    def body(x_ref, win0_ref, wout0_ref, win1_ref, wout1_ref, win2_ref,
             wout2_ref, out_ref, x_buf, comm_ref, send_sems, recv_sems):
        my = lax.axis_index("i")
        left = (my + N_DEV - 1) % N_DEV
        right = (my + 1) % N_DEV

        barrier_sem = pltpu.get_barrier_semaphore()
        for nbr in (left, right):
            pl.semaphore_signal(barrier_sem, inc=1, device_id=(nbr,),
                                device_id_type=pl.DeviceIdType.MESH)
        pl.semaphore_wait(barrier_sem, 2)

        x_buf[:, :] = x_ref[:, :]

        wins = [win0_ref, win1_ref, win2_ref]
        wouts = [wout0_ref, wout1_ref, wout2_ref]
        for layer in range(N_LAYERS):
            h = jnp.maximum(
                jnp.dot(x_buf[:, :], wins[layer][:, :],
                        preferred_element_type=jnp.float32), 0.0)
            partial = jnp.dot(h, wouts[layer][:, :],
                              preferred_element_type=jnp.float32)
            comm_ref[0, :, :] = partial
            acc = partial
            for hop in range(N_DEV - 1):
                rdma = pltpu.make_async_remote_copy(
                    src_ref=comm_ref.at[hop],
                    dst_ref=comm_ref.at[hop + 1],
                    send_sem=send_sems.at[layer, hop],
                    recv_sem=recv_sems.at[layer, hop],
                    device_id=(right,),
                    device_id_type=pl.DeviceIdType.MESH,
                )
                rdma.start()
                rdma.wait()
                acc = acc + comm_ref[hop + 1, :, :]
            x_buf[:, :] = acc
            if layer < N_LAYERS - 1:
                for nbr in (left, right):
                    pl.semaphore_signal(barrier_sem, inc=1, device_id=(nbr,),
                                        device_id_type=pl.DeviceIdType.MESH)
                pl.semaphore_wait(barrier_sem, 2)

        out_ref[:, :] = x_buf[pl.ds(my * rows_per, rows_per), :]

    return pl.pallas_call(
        body,
        out_shape=jax.ShapeDtypeStruct((rows_per, d), jnp.float32),
        in_specs=[pl.BlockSpec(memory_space=pltpu.VMEM)] * 7,
        out_specs=pl.BlockSpec(memory_space=pltpu.VMEM),
        scratch_shapes=[
            pltpu.VMEM((b, d), jnp.float32),
            pltpu.VMEM((N_DEV, b, d), jnp.float32),
            pltpu.SemaphoreType.DMA((N_LAYERS, N_DEV - 1)),
            pltpu.SemaphoreType.DMA((N_LAYERS, N_DEV - 1)),
        ],
        compiler_params=pltpu.CompilerParams(collective_id=0),
    )(x, Win0, Wout0, Win1, Wout1, Win2, Wout2)


# baseline (device time: 69437 ns/iter reference)
import jax
import jax.numpy as jnp
from jax import lax
from jax.experimental import pallas as pl
from jax.experimental.pallas import tpu as pltpu

N_DEV = 4
N_LAYERS = 3


def kernel(x, Win0, Wout0, Win1, Wout1, Win2, Wout2):
    b, d = x.shape
    rows_per = b // N_DEV

    def body(x_ref, win0_ref, wout0_ref, win1_ref, wout1_ref, win2_ref,
             wout2_ref, out_ref, x_buf, comm_ref, send_sems, recv_sems):
        my = lax.axis_index("i")
        left = (my + N_DEV - 1) % N_DEV
        right = (my + 1) % N_DEV

        barrier_sem = pltpu.get_barrier_semaphore()
        for nbr in (left, right):
            pl.semaphore_signal(barrier_sem, inc=1, device_id=(nbr,),
                                device_id_type=pl.DeviceIdType.MESH)
        pl.semaphore_wait(barrier_sem, 2)

        x_buf[:, :] = x_ref[:, :]

        wins = [win0_ref, win1_ref, win2_ref]
        wouts = [wout0_ref, wout1_ref, wout2_ref]
        for layer in range(N_LAYERS):
            h = jnp.maximum(
                jnp.dot(x_buf[:, :], wins[layer][:, :],
                        preferred_element_type=jnp.float32), 0.0)
            partial = jnp.dot(h, wouts[layer][:, :],
                              preferred_element_type=jnp.float32)
            comm_ref[0, :, :] = partial
            acc = partial
            for hop in range(N_DEV - 1):
                rdma = pltpu.make_async_remote_copy(
                    src_ref=comm_ref.at[hop],
                    dst_ref=comm_ref.at[hop + 1],
                    send_sem=send_sems.at[layer, hop],
                    recv_sem=recv_sems.at[layer, hop],
                    device_id=(right,),
                    device_id_type=pl.DeviceIdType.MESH,
                )
                rdma.start()
                rdma.wait()
                acc = acc + comm_ref[hop + 1, :, :]
            x_buf[:, :] = acc
            if layer < N_LAYERS - 1:
                for nbr in (left, right):
                    pl.semaphore_signal(barrier_sem, inc=1, device_id=(nbr,),
                                        device_id_type=pl.DeviceIdType.MESH)
                pl.semaphore_wait(barrier_sem, 2)

        out_ref[:, :] = x_buf[pl.ds(my * rows_per, rows_per), :]

    return pl.pallas_call(
        body,
        out_shape=jax.ShapeDtypeStruct((rows_per, d), jnp.float32),
        in_specs=[pl.BlockSpec(memory_space=pltpu.VMEM)] * 7,
        out_specs=pl.BlockSpec(memory_space=pltpu.VMEM),
        scratch_shapes=[
            pltpu.VMEM((b, d), jnp.float32),
            pltpu.VMEM((N_DEV, b, d), jnp.float32),
            pltpu.SemaphoreType.DMA((N_LAYERS, N_DEV - 1)),
            pltpu.SemaphoreType.DMA((N_LAYERS, N_DEV - 1)),
        ],
        compiler_params=pltpu.CompilerParams(
            collective_id=0,
            vmem_limit_bytes=100 * 1024 * 1024,
        ),
    )(x, Win0, Wout0, Win1, Wout1, Win2, Wout2)


# device time: 43607 ns/iter; 1.5923x vs baseline; 1.5923x over previous
import jax
import jax.numpy as jnp
from jax import lax
from jax.experimental import pallas as pl
from jax.experimental.pallas import tpu as pltpu

N_DEV = 4
N_LAYERS = 3
N_PHASES = 2 * N_LAYERS - 1


def kernel(x, Win0, Wout0, Win1, Wout1, Win2, Wout2):
    b, d = x.shape
    rows_per = b // N_DEV

    def body(x_ref, win0_ref, wout0_ref, win1_ref, wout1_ref, win2_ref,
             wout2_ref, out_ref, x_buf, part_ref, red_ref, rs_ref,
             send_sems, recv_sems):
        my = lax.axis_index("i")

        barrier_sem = pltpu.get_barrier_semaphore()
        for k in range(1, N_DEV):
            pl.semaphore_signal(barrier_sem, inc=1,
                                device_id=((my + k) % N_DEV,),
                                device_id_type=pl.DeviceIdType.MESH)
        pl.semaphore_wait(barrier_sem, N_DEV - 1)

        wins = [win0_ref, win1_ref, win2_ref]
        wouts = [wout0_ref, wout1_ref, wout2_ref]
        for layer in range(N_LAYERS):
            xin = x_ref if layer == 0 else x_buf
            h = jnp.maximum(
                jnp.dot(xin[:, :], wins[layer][:, :],
                        preferred_element_type=jnp.float32), 0.0)
            part_ref[:, :] = jnp.dot(h, wouts[layer][:, :],
                                     preferred_element_type=jnp.float32)

            phase = 2 * layer
            rdmas = []
            for k in range(1, N_DEV):
                r = (my + k) % N_DEV
                s = N_DEV - 1 - k
                rdma = pltpu.make_async_remote_copy(
                    src_ref=part_ref.at[pl.ds(r * rows_per, rows_per), :],
                    dst_ref=rs_ref.at[s],
                    send_sem=send_sems.at[phase, s],
                    recv_sem=recv_sems.at[phase, s],
                    device_id=(r,),
                    device_id_type=pl.DeviceIdType.MESH,
                )
                rdma.start()
                rdmas.append(rdma)
            for rdma in rdmas:
                rdma.wait()

            reduced = (part_ref[pl.ds(my * rows_per, rows_per), :]
                       + rs_ref[0] + rs_ref[1] + rs_ref[2])
            if layer == N_LAYERS - 1:
                out_ref[:, :] = reduced
                break

            red_ref[:, :] = reduced
            x_buf[pl.ds(my * rows_per, rows_per), :] = reduced
            phase = 2 * layer + 1
            rdmas = []
            for k in range(1, N_DEV):
                r = (my + k) % N_DEV
                s = N_DEV - 1 - k
                rdma = pltpu.make_async_remote_copy(
                    src_ref=red_ref,
                    dst_ref=x_buf.at[pl.ds(my * rows_per, rows_per), :],
                    send_sem=send_sems.at[phase, s],
                    recv_sem=recv_sems.at[phase, s],
                    device_id=(r,),
                    device_id_type=pl.DeviceIdType.MESH,
                )
                rdma.start()
                rdmas.append(rdma)
            for rdma in rdmas:
                rdma.wait()

    return pl.pallas_call(
        body,
        out_shape=jax.ShapeDtypeStruct((rows_per, d), jnp.float32),
        in_specs=[pl.BlockSpec(memory_space=pltpu.VMEM)] * 7,
        out_specs=pl.BlockSpec(memory_space=pltpu.VMEM),
        scratch_shapes=[
            pltpu.VMEM((b, d), jnp.float32),
            pltpu.VMEM((b, d), jnp.float32),
            pltpu.VMEM((rows_per, d), jnp.float32),
            pltpu.VMEM((N_DEV - 1, rows_per, d), jnp.float32),
            pltpu.SemaphoreType.DMA((N_PHASES, N_DEV - 1)),
            pltpu.SemaphoreType.DMA((N_PHASES, N_DEV - 1)),
        ],
        compiler_params=pltpu.CompilerParams(
            collective_id=0,
            vmem_limit_bytes=100 * 1024 * 1024,
        ),
    )(x, Win0, Wout0, Win1, Wout1, Win2, Wout2)
